# baseline (speedup 1.0000x reference)
import jax
import jax.numpy as jnp
from jax import lax
from jax.experimental import pallas as pl
from jax.experimental.pallas import tpu as pltpu
from jax.experimental.pallas import tpu_sc as plsc

SRC_ROWS = 1000000
N_ROWS = 16384
N_COLS = 64
B = N_ROWS * N_COLS
NW = 32
W_ELEMS = B // NW
ROWS_PER_W = N_ROWS // NW
L = 16
QS = N_COLS // L

FLAT_BLK_ROWS = 4000
FLAT_GRID = SRC_ROWS // FLAT_BLK_ROWS


def _flatten_body(src_ref, out_ref):
    x = src_ref[...].reshape(FLAT_BLK_ROWS // 2, 2, N_COLS)
    out_ref[...] = jnp.concatenate([x[:, 0, :], x[:, 1, :]], axis=1)


def _flatten(source):
    return pl.pallas_call(
        _flatten_body,
        grid=(FLAT_GRID,),
        in_specs=[pl.BlockSpec((FLAT_BLK_ROWS, N_COLS), lambda i: (i, 0))],
        out_specs=pl.BlockSpec(
            (FLAT_BLK_ROWS // 2, 2 * N_COLS), lambda i: (i, 0)),
        out_shape=jax.ShapeDtypeStruct(
            (SRC_ROWS // 2, 2 * N_COLS), jnp.float32),
        compiler_params=pltpu.CompilerParams(
            dimension_semantics=("arbitrary",)),
    )(source)


def _gather_body(src_hbm, idx_hbm, out_hbm, a_v, flat_v, g_v, sem):
    c = lax.axis_index("c")
    s = lax.axis_index("s")
    wid = s * 2 + c
    rb = wid * ROWS_PER_W

    a_i32 = a_v.bitcast(jnp.int32)

    pltpu.sync_copy(idx_hbm.at[pl.ds(rb, ROWS_PER_W)], a_i32)

    lane = lax.iota(jnp.int32, L)

    def arith(r, carry):
        for q in range(QS):
            v = a_i32[r, pl.ds(q * L, L)]
            flat_v[pl.ds(r * N_COLS + q * L, L)] = v * N_COLS + q * L + lane
        return carry

    lax.fori_loop(0, ROWS_PER_W, arith, 0)

    pltpu.async_copy(src_hbm.at[flat_v], g_v, sem).wait()

    def repack(r, carry):
        for q in range(QS):
            a_v[r, pl.ds(q * L, L)] = g_v[pl.ds(r * N_COLS + q * L, L)]
        return carry

    lax.fori_loop(0, ROWS_PER_W, repack, 0)

    pltpu.sync_copy(a_v, out_hbm.at[pl.ds(rb, ROWS_PER_W)])


def kernel(source, source_idx_2d):
    src_flat = _flatten(source).reshape(-1)
    idx = source_idx_2d.astype(jnp.int32)
    mesh = plsc.VectorSubcoreMesh(core_axis_name="c", subcore_axis_name="s")
    return pl.kernel(
        _gather_body,
        out_type=jax.ShapeDtypeStruct((N_ROWS, N_COLS), jnp.float32),
        mesh=mesh,
        scratch_types=[
            pltpu.VMEM((ROWS_PER_W, N_COLS), jnp.float32),
            pltpu.VMEM((W_ELEMS,), jnp.int32),
            pltpu.VMEM((W_ELEMS,), jnp.float32),
            pltpu.SemaphoreType.DMA,
        ],
    )(src_flat, idx)


# speedup vs baseline: 1.2642x; 1.2642x over previous
import jax
import jax.numpy as jnp
from jax import lax
from jax.experimental import pallas as pl
from jax.experimental.pallas import tpu as pltpu
from jax.experimental.pallas import tpu_sc as plsc

SRC_ROWS = 1000000
N_ROWS = 16384
N_COLS = 64
B = N_ROWS * N_COLS
NW = 32
W_ELEMS = B // NW
ROWS_PER_W = N_ROWS // NW
L = 16
QS = N_COLS // L


def _gather_body(src_hbm, idx_hbm, out_hbm, a_v, flat_v, g_v, sem):
    c = lax.axis_index("c")
    s = lax.axis_index("s")
    wid = s * 2 + c
    rb = wid * ROWS_PER_W

    a_i32 = a_v.bitcast(jnp.int32)

    pltpu.sync_copy(idx_hbm.at[pl.ds(rb, ROWS_PER_W)], a_i32)

    lane = lax.iota(jnp.int32, L)

    def arith(r, carry):
        for q in range(QS):
            v = a_i32[r, pl.ds(q * L, L)]
            flat_v[pl.ds(r * N_COLS + q * L, L)] = v * N_COLS + q * L + lane
        return carry

    lax.fori_loop(0, ROWS_PER_W, arith, 0)

    pltpu.async_copy(src_hbm.at[flat_v], g_v, sem).wait()

    def repack(r, carry):
        for q in range(QS):
            a_v[r, pl.ds(q * L, L)] = g_v[pl.ds(r * N_COLS + q * L, L)]
        return carry

    lax.fori_loop(0, ROWS_PER_W, repack, 0)

    pltpu.sync_copy(a_v, out_hbm.at[pl.ds(rb, ROWS_PER_W)])


def kernel(source, source_idx_2d):
    depadded = jax.lax.optimization_barrier(source.reshape(SRC_ROWS // 2,
                                                           2 * N_COLS))
    src_flat = depadded.reshape(-1)
    idx = source_idx_2d.astype(jnp.int32)
    mesh = plsc.VectorSubcoreMesh(core_axis_name="c", subcore_axis_name="s")
    return pl.kernel(
        _gather_body,
        out_type=jax.ShapeDtypeStruct((N_ROWS, N_COLS), jnp.float32),
        mesh=mesh,
        scratch_types=[
            pltpu.VMEM((ROWS_PER_W, N_COLS), jnp.float32),
            pltpu.VMEM((W_ELEMS,), jnp.int32),
            pltpu.VMEM((W_ELEMS,), jnp.float32),
            pltpu.SemaphoreType.DMA,
        ],
    )(src_flat, idx)


# speedup vs baseline: 1.2834x; 1.0152x over previous
import jax
import jax.numpy as jnp
from jax import lax
from jax.experimental import pallas as pl
from jax.experimental.pallas import tpu as pltpu
from jax.experimental.pallas import tpu_sc as plsc

SRC_ROWS = 1000000
N_ROWS = 16384
N_COLS = 64
B = N_ROWS * N_COLS
NW = 32
W_ELEMS = B // NW
ROWS_PER_W = N_ROWS // NW
L = 16
QS = N_COLS // L
NCH = 8
CH_ROWS = ROWS_PER_W // NCH
CH_ELEMS = CH_ROWS * N_COLS


def _gather_body(src_hbm, idx_hbm, out_hbm, a_v, flat_v, ga_v, gb_v,
                 sema, semb):
    c = lax.axis_index("c")
    s = lax.axis_index("s")
    wid = s * 2 + c
    rb = wid * ROWS_PER_W

    a_i32 = a_v.bitcast(jnp.int32)
    pltpu.sync_copy(idx_hbm.at[pl.ds(rb, ROWS_PER_W)], a_i32)

    lane = lax.iota(jnp.int32, L)
    qofs = [lane + q * L for q in range(QS)]

    def arith_chunk(k):
        def body(r0, carry):
            r = k * CH_ROWS + r0
            for q in range(QS):
                v = a_i32[r, pl.ds(q * L, L)]
                flat_v[pl.ds(r * N_COLS + q * L, L)] = v * N_COLS + qofs[q]
            return carry
        lax.fori_loop(0, CH_ROWS, body, 0)

    def fire(k, buf, sem):
        return pltpu.async_copy(
            src_hbm.at[flat_v.at[pl.ds(k * CH_ELEMS, CH_ELEMS)]], buf, sem)

    def repack_chunk(k, buf):
        def body(r0, carry):
            for q in range(QS):
                a_v[k * CH_ROWS + r0, pl.ds(q * L, L)] = (
                    buf[pl.ds(r0 * N_COLS + q * L, L)])
            return carry
        lax.fori_loop(0, CH_ROWS, body, 0)

    bufs = [(ga_v, sema), (gb_v, semb)]
    cps = {}
    arith_chunk(0)
    cps[0] = fire(0, *bufs[0])
    arith_chunk(1)
    cps[1] = fire(1, *bufs[1])
    for k in range(2, NCH):
        buf, sem = bufs[k % 2]
        cps[k - 2].wait()
        repack_chunk(k - 2, buf)
        arith_chunk(k)
        cps[k] = fire(k, buf, sem)
    cps[NCH - 2].wait()
    repack_chunk(NCH - 2, bufs[NCH % 2][0])
    cps[NCH - 1].wait()
    repack_chunk(NCH - 1, bufs[(NCH + 1) % 2][0])

    pltpu.sync_copy(a_v, out_hbm.at[pl.ds(rb, ROWS_PER_W)])


def kernel(source, source_idx_2d):
    src_flat = source.reshape(-1)
    idx = source_idx_2d.astype(jnp.int32)
    mesh = plsc.VectorSubcoreMesh(core_axis_name="c", subcore_axis_name="s")
    return pl.kernel(
        _gather_body,
        out_type=jax.ShapeDtypeStruct((N_ROWS, N_COLS), jnp.float32),
        mesh=mesh,
        scratch_types=[
            pltpu.VMEM((ROWS_PER_W, N_COLS), jnp.float32),
            pltpu.VMEM((W_ELEMS,), jnp.int32),
            pltpu.VMEM((CH_ELEMS,), jnp.float32),
            pltpu.VMEM((CH_ELEMS,), jnp.float32),
            pltpu.SemaphoreType.DMA,
            pltpu.SemaphoreType.DMA,
        ],
    )(src_flat, idx)
